# Initial kernel scaffold; baseline (speedup 1.0000x reference)
import jax
import jax.numpy as jnp
from jax.experimental import pallas as pl


def kernel(in_features, conv_w, conv_b, fc1_w, fc1_b, fc2_w, fc2_b, fco_w, fco_b):
    raise NotImplementedError("write your pallas kernel here")


# speedup vs baseline: 1.2392x; 1.2392x over previous
import functools

import jax
import jax.numpy as jnp
from jax.experimental import pallas as pl
from jax.experimental.pallas import tpu as pltpu

B, C, N, R = 8, 2048, 8192, 5
CHUNK_N = 512


def _proj_body(x_ref, w_ref, out_ref):
    x = x_ref[0]
    w = w_ref[...]
    out_ref[0] = jnp.sum(x * w, axis=0, keepdims=True)


def _project(in_features, conv_w):
    w_col = conv_w.reshape(C, 1)
    return pl.pallas_call(
        _proj_body,
        grid=(B, N // CHUNK_N),
        in_specs=[
            pl.BlockSpec((1, C, CHUNK_N), lambda b, n: (b, 0, n)),
            pl.BlockSpec((C, 1), lambda b, n: (0, 0)),
        ],
        out_specs=pl.BlockSpec((1, 1, CHUNK_N), lambda b, n: (b, 0, n)),
        out_shape=jax.ShapeDtypeStruct((B, 1, N), jnp.float32),
        compiler_params=pltpu.CompilerParams(
            dimension_semantics=("parallel", "parallel"),
        ),
    )(in_features, w_col)


def _head_body(agg_ref, b0_ref, w1_ref, b1_ref, w2_ref, b2_ref, wo_ref,
               bo_ref, out_ref):
    a = agg_ref[...]
    idx = jax.lax.broadcasted_iota(jnp.int32, (B, N), 1)
    lane = jax.lax.broadcasted_iota(jnp.int32, (B, 16), 1)
    mil = jnp.zeros((B, 16), jnp.float32)

    work = a
    for r in range(R):
        m = jnp.max(work, axis=1, keepdims=True)
        mil = jnp.where(lane == r, m, mil)
        first = jnp.min(jnp.where(work == m, idx, N), axis=1, keepdims=True)
        work = jnp.where(idx == first, -jnp.inf, work)
    work = a
    for r in range(R):
        m = jnp.min(work, axis=1, keepdims=True)
        mil = jnp.where(lane == R + r, m, mil)
        first = jnp.min(jnp.where(work == m, idx, N), axis=1, keepdims=True)
        work = jnp.where(idx == first, jnp.inf, work)

    mil = mil + b0_ref[0, 0]
    x = jax.nn.sigmoid(
        jnp.dot(mil, w1_ref[...], preferred_element_type=jnp.float32)
        + b1_ref[...])
    x = jax.nn.sigmoid(
        jnp.dot(x, w2_ref[...], preferred_element_type=jnp.float32)
        + b2_ref[...])
    out_ref[...] = jax.nn.sigmoid(
        jnp.dot(x, wo_ref[...], preferred_element_type=jnp.float32)
        + bo_ref[...])


def _head(agg, conv_b, fc1_w, fc1_b, fc2_w, fc2_b, fco_w, fco_b):
    n1, n2 = fc1_w.shape[0], fc2_w.shape[0]
    w1 = jnp.zeros((16, n1), jnp.float32).at[:2 * R].set(fc1_w.T)
    return pl.pallas_call(
        _head_body,
        out_shape=jax.ShapeDtypeStruct((B, 1), jnp.float32),
    )(agg, conv_b.reshape(1, 1), w1, fc1_b.reshape(1, n1), fc2_w.T,
      fc2_b.reshape(1, n2), fco_w.T, fco_b.reshape(1, 1))


def kernel(in_features, conv_w, conv_b, fc1_w, fc1_b, fc2_w, fc2_b, fco_w,
           fco_b):
    agg = _project(in_features, conv_w).reshape(B, N)
    out = _head(agg, conv_b, fc1_w, fc1_b, fc2_w, fc2_b, fco_w, fco_b)
    return out.reshape(B, 1, 1)


# speedup vs baseline: 1.3644x; 1.1011x over previous
import functools

import jax
import jax.numpy as jnp
from jax.experimental import pallas as pl
from jax.experimental.pallas import tpu as pltpu

B, C, N, R = 8, 2048, 8192, 5
CHUNK_N = 1024


def _proj_body(x_ref, w_ref, out_ref):
    x = x_ref[0]
    w = w_ref[...]
    out_ref[0] = jnp.sum(x * w, axis=0, keepdims=True)


def _project(in_features, conv_w):
    w_col = conv_w.reshape(C, 1)
    return pl.pallas_call(
        _proj_body,
        grid=(B, N // CHUNK_N),
        in_specs=[
            pl.BlockSpec((1, C, CHUNK_N), lambda b, n: (b, 0, n)),
            pl.BlockSpec((C, 1), lambda b, n: (0, 0)),
        ],
        out_specs=pl.BlockSpec((1, 1, CHUNK_N), lambda b, n: (b, 0, n)),
        out_shape=jax.ShapeDtypeStruct((B, 1, N), jnp.float32),
        compiler_params=pltpu.CompilerParams(
            dimension_semantics=("parallel", "parallel"),
        ),
    )(in_features, w_col)


def _head_body(agg_ref, b0_ref, w1_ref, b1_ref, w2_ref, b2_ref, wo_ref,
               bo_ref, out_ref):
    a = agg_ref[...]
    idx = jax.lax.broadcasted_iota(jnp.int32, (B, N), 1)
    lane = jax.lax.broadcasted_iota(jnp.int32, (B, 16), 1)
    mil = jnp.zeros((B, 16), jnp.float32)

    work = a
    for r in range(R):
        m = jnp.max(work, axis=1, keepdims=True)
        mil = jnp.where(lane == r, m, mil)
        first = jnp.min(jnp.where(work == m, idx, N), axis=1, keepdims=True)
        work = jnp.where(idx == first, -jnp.inf, work)
    work = a
    for r in range(R):
        m = jnp.min(work, axis=1, keepdims=True)
        mil = jnp.where(lane == R + r, m, mil)
        first = jnp.min(jnp.where(work == m, idx, N), axis=1, keepdims=True)
        work = jnp.where(idx == first, jnp.inf, work)

    mil = mil + b0_ref[0, 0]
    x = jax.nn.sigmoid(
        jnp.dot(mil, w1_ref[...], preferred_element_type=jnp.float32)
        + b1_ref[...])
    x = jax.nn.sigmoid(
        jnp.dot(x, w2_ref[...], preferred_element_type=jnp.float32)
        + b2_ref[...])
    out_ref[...] = jax.nn.sigmoid(
        jnp.dot(x, wo_ref[...], preferred_element_type=jnp.float32)
        + bo_ref[...])


def _head(agg, conv_b, fc1_w, fc1_b, fc2_w, fc2_b, fco_w, fco_b):
    n1, n2 = fc1_w.shape[0], fc2_w.shape[0]
    w1 = jnp.zeros((16, n1), jnp.float32).at[:2 * R].set(fc1_w.T)
    return pl.pallas_call(
        _head_body,
        out_shape=jax.ShapeDtypeStruct((B, 1), jnp.float32),
    )(agg, conv_b.reshape(1, 1), w1, fc1_b.reshape(1, n1), fc2_w.T,
      fc2_b.reshape(1, n2), fco_w.T, fco_b.reshape(1, 1))


def kernel(in_features, conv_w, conv_b, fc1_w, fc1_b, fc2_w, fc2_b, fco_w,
           fco_b):
    agg = _project(in_features, conv_w).reshape(B, N)
    out = _head(agg, conv_b, fc1_w, fc1_b, fc2_w, fc2_b, fco_w, fco_b)
    return out.reshape(B, 1, 1)
